# baseline (speedup 1.0000x reference)
import functools

import jax
import jax.numpy as jnp
from jax import lax
from jax.experimental import pallas as pl
from jax.experimental.pallas import tpu as pltpu
from jax.experimental.pallas import tpu_sc as plsc

BATCH = 16384
D = 64
NC = 2
NS = 16
NW = NC * NS
BPW = BATCH // NW
PASS = 128
NPASS = BPW // PASS

_mesh = plsc.VectorSubcoreMesh(core_axis_name="c", subcore_axis_name="s")

_GATHER_DNUMS = lax.GatherDimensionNumbers(
    offset_dims=(), collapsed_slice_dims=(0,), start_index_map=(0,))


def _permute(x, idx):
    return lax.gather(x, idx[:, None], _GATHER_DNUMS, (1,),
                      mode=lax.GatherScatterMode.PROMISE_IN_BOUNDS)


def _extract(v, r):
    return lax.squeeze(lax.slice(v, (r,), (r + 1,)), (0,))


@functools.partial(
    pl.kernel,
    out_type=jax.ShapeDtypeStruct((BATCH,), jnp.float32),
    mesh=_mesh,
    scratch_types=[
        pltpu.VMEM((BPW,), jnp.int32),
        pltpu.VMEM((BPW,), jnp.int32),
        pltpu.VMEM((NPASS, PASS), jnp.int32),
        pltpu.VMEM((NPASS, PASS), jnp.int32),
        pltpu.VMEM((BPW,), jnp.int32),
        pltpu.VMEM((BPW,), jnp.int32),
        pltpu.VMEM((2, PASS, 2 * D), jnp.float32),
        pltpu.VMEM((2, PASS, 2 * D), jnp.float32),
        pltpu.VMEM((BPW,), jnp.float32),
        pltpu.SemaphoreType.DMA,
        pltpu.SemaphoreType.DMA,
    ],
)
def _mf_sc(u_hbm, i_hbm, ue_hbm, ie_hbm, out_hbm,
           uraw_v, iraw_v, uidx_v, iidx_v, uoff_v, ioff_v,
           pu_v, qi_v, out_v, sem0, sem1):
    wid = lax.axis_index("s") * NC + lax.axis_index("c")
    base = wid * BPW

    pltpu.sync_copy(u_hbm.at[pl.ds(base, BPW)], uraw_v)
    pltpu.sync_copy(i_hbm.at[pl.ds(base, BPW)], iraw_v)

    for k in range(BPW // 16):
        c, o = k // (PASS // 16), (k % (PASS // 16)) * 16
        uv = uraw_v[pl.ds(k * 16, 16)]
        iv = iraw_v[pl.ds(k * 16, 16)]
        uidx_v[c, pl.ds(o, 16)] = lax.shift_right_logical(uv, 1)
        iidx_v[c, pl.ds(o, 16)] = lax.shift_right_logical(iv, 1)
        uoff_v[pl.ds(k * 16, 16)] = lax.shift_left(uv & 1, 6)
        ioff_v[pl.ds(k * 16, 16)] = lax.shift_left(iv & 1, 6)

    sems = (sem0, sem1)

    def fire(p):
        s = sems[p % 2]
        cu = pltpu.async_copy(ue_hbm.at[uidx_v.at[p]], pu_v.at[p % 2], s)
        ci = pltpu.async_copy(ie_hbm.at[iidx_v.at[p]], qi_v.at[p % 2], s)
        return cu, ci

    lanes = lax.iota(jnp.int32, 16)
    rots = [(lanes + s) % 16 for s in (8, 4, 2, 1)]

    def make_group_body(p):
        slot = p % 2

        def group_body(g, carry):
            b0 = g * 16
            uo = uoff_v[pl.ds(p * PASS + b0, 16)]
            io = ioff_v[pl.ds(p * PASS + b0, 16)]
            tot = jnp.zeros((16,), jnp.float32)
            for r in range(16):
                b = b0 + r
                ou = _extract(uo, r)
                oi = _extract(io, r)
                acc = (pu_v[slot, b, pl.ds(ou, 16)] *
                       qi_v[slot, b, pl.ds(oi, 16)])
                for j in range(1, D // 16):
                    acc = acc + (pu_v[slot, b, pl.ds(ou + j * 16, 16)] *
                                 qi_v[slot, b, pl.ds(oi + j * 16, 16)])
                for idx in rots:
                    acc = acc + _permute(acc, idx)
                tot = jnp.where(lanes == r, acc, tot)
            out_v[pl.ds(p * PASS + b0, 16)] = tot
            return carry

        return group_body

    pending = fire(0)
    for p in range(NPASS):
        nxt = fire(p + 1) if p + 1 < NPASS else None
        for cp in pending:
            cp.wait()
        lax.fori_loop(0, PASS // 16, make_group_body(p), 0)
        pending = nxt

    pltpu.sync_copy(out_v, out_hbm.at[pl.ds(base, BPW)])


def kernel(u, i, user_emb, item_emb):
    ue2 = user_emb.reshape(user_emb.shape[0] // 2, 2 * D)
    ie2 = item_emb.reshape(item_emb.shape[0] // 2, 2 * D)
    return _mf_sc(u, i, ue2, ie2)


# speedup vs baseline: 1.0690x; 1.0690x over previous
import functools

import jax
import jax.numpy as jnp
from jax import lax
from jax.experimental import pallas as pl
from jax.experimental.pallas import tpu as pltpu
from jax.experimental.pallas import tpu_sc as plsc

BATCH = 16384
D = 64
W = 128
NC = 2
NS = 16
NW = NC * NS
BPW = BATCH // NW
PASS = 128
NPASS = BPW // PASS

_mesh = plsc.VectorSubcoreMesh(core_axis_name="c", subcore_axis_name="s")

_GATHER_DNUMS = lax.GatherDimensionNumbers(
    offset_dims=(), collapsed_slice_dims=(0,), start_index_map=(0,))


def _permute(x, idx):
    return lax.gather(x, idx[:, None], _GATHER_DNUMS, (1,),
                      mode=lax.GatherScatterMode.PROMISE_IN_BOUNDS)


@functools.partial(
    pl.kernel,
    out_type=jax.ShapeDtypeStruct((BATCH,), jnp.float32),
    mesh=_mesh,
    scratch_types=[
        pltpu.VMEM((NPASS, PASS), jnp.int32),
        pltpu.VMEM((NPASS, PASS), jnp.int32),
        pltpu.VMEM((2, PASS, W), jnp.float32),
        pltpu.VMEM((2, PASS, W), jnp.float32),
        pltpu.VMEM((BPW,), jnp.float32),
        pltpu.SemaphoreType.DMA,
        pltpu.SemaphoreType.DMA,
    ],
)
def _mf_sc(u_hbm, i_hbm, ue_hbm, ie_hbm, out_hbm,
           uidx_v, iidx_v, pu_v, qi_v, out_v, sem0, sem1):
    wid = lax.axis_index("s") * NC + lax.axis_index("c")
    base = wid * BPW

    for p in range(NPASS):
        pltpu.sync_copy(u_hbm.at[pl.ds(base + p * PASS, PASS)], uidx_v.at[p])
        pltpu.sync_copy(i_hbm.at[pl.ds(base + p * PASS, PASS)], iidx_v.at[p])

    sems = (sem0, sem1)

    def fire(p):
        s = sems[p % 2]
        cu = pltpu.async_copy(ue_hbm.at[uidx_v.at[p]], pu_v.at[p % 2], s)
        ci = pltpu.async_copy(ie_hbm.at[iidx_v.at[p]], qi_v.at[p % 2], s)
        return cu, ci

    lanes = lax.iota(jnp.int32, 16)
    rots = [(lanes + s) % 16 for s in (8, 4, 2, 1)]

    def make_group_body(p):
        slot = p % 2

        def group_body(g, carry):
            b0 = g * 16
            tot = jnp.zeros((16,), jnp.float32)
            for r in range(16):
                b = b0 + r
                acc = pu_v[slot, b, pl.ds(0, 16)] * qi_v[slot, b, pl.ds(0, 16)]
                for j in range(1, D // 16):
                    acc = acc + (pu_v[slot, b, pl.ds(j * 16, 16)] *
                                 qi_v[slot, b, pl.ds(j * 16, 16)])
                for idx in rots:
                    acc = acc + _permute(acc, idx)
                tot = jnp.where(lanes == r, acc, tot)
            out_v[pl.ds(p * PASS + b0, 16)] = tot
            return carry

        return group_body

    pending = fire(0)
    for p in range(NPASS):
        nxt = fire(p + 1) if p + 1 < NPASS else None
        for cp in pending:
            cp.wait()
        lax.fori_loop(0, PASS // 16, make_group_body(p), 0)
        pending = nxt

    pltpu.sync_copy(out_v, out_hbm.at[pl.ds(base, BPW)])


def kernel(u, i, user_emb, item_emb):
    uep = jnp.pad(user_emb, ((0, 0), (0, W - D)))
    iep = jnp.pad(item_emb, ((0, 0), (0, W - D)))
    return _mf_sc(u, i, uep, iep)


# speedup vs baseline: 1.4919x; 1.3956x over previous
import functools

import jax
import jax.numpy as jnp
from jax import lax
from jax.experimental import pallas as pl
from jax.experimental.pallas import tpu as pltpu
from jax.experimental.pallas import tpu_sc as plsc

BATCH = 16384
D = 64
NC = 2
NS = 16
NW = NC * NS
BPW = BATCH // NW
G = 16
NG = BPW // G

_mesh = plsc.VectorSubcoreMesh(core_axis_name="c", subcore_axis_name="s")

_GATHER_DNUMS = lax.GatherDimensionNumbers(
    offset_dims=(), collapsed_slice_dims=(0,), start_index_map=(0,))


def _permute(x, idx):
    return lax.gather(x, idx[:, None], _GATHER_DNUMS, (1,),
                      mode=lax.GatherScatterMode.PROMISE_IN_BOUNDS)


def _extract(v, r):
    return lax.squeeze(lax.slice(v, (r,), (r + 1,)), (0,))


@functools.partial(
    pl.kernel,
    out_type=jax.ShapeDtypeStruct((BATCH,), jnp.float32),
    mesh=_mesh,
    scratch_types=[
        pltpu.VMEM((BPW,), jnp.int32),
        pltpu.VMEM((BPW,), jnp.int32),
        pltpu.VMEM((2, G, 8, D), jnp.float32),
        pltpu.VMEM((2, G, 8, D), jnp.float32),
        pltpu.VMEM((BPW,), jnp.float32),
        pltpu.SemaphoreType.DMA,
        pltpu.SemaphoreType.DMA,
    ],
)
def _mf_sc(u_hbm, i_hbm, ue_hbm, ie_hbm, out_hbm,
           uraw_v, iraw_v, pu_v, qi_v, out_v, sem0, sem1):
    wid = lax.axis_index("s") * NC + lax.axis_index("c")
    base = wid * BPW

    pltpu.sync_copy(u_hbm.at[pl.ds(base, BPW)], uraw_v)
    pltpu.sync_copy(i_hbm.at[pl.ds(base, BPW)], iraw_v)

    sems = (sem0, sem1)
    lanes = lax.iota(jnp.int32, 16)
    rots = [(lanes + s) % 16 for s in (8, 4, 2, 1)]

    def fire(g, slot):
        s = sems[slot]
        uvec = uraw_v[pl.ds(g * G, G)]
        ivec = iraw_v[pl.ds(g * G, G)]
        for r in range(G):
            ub = lax.shift_right_logical(_extract(uvec, r), 3) * 8
            ib = lax.shift_right_logical(_extract(ivec, r), 3) * 8
            pltpu.async_copy(ue_hbm.at[pl.ds(ub, 8), :], pu_v.at[slot, r], s)
            pltpu.async_copy(ie_hbm.at[pl.ds(ib, 8), :], qi_v.at[slot, r], s)

    def drain(slot):
        s = sems[slot]
        for r in range(G):
            pltpu.make_async_copy(ue_hbm.at[pl.ds(0, 8), :], pu_v.at[slot, r], s).wait()
            pltpu.make_async_copy(ie_hbm.at[pl.ds(0, 8), :], qi_v.at[slot, r], s).wait()

    def compute(g, slot):
        uvec = uraw_v[pl.ds(g * G, G)]
        ivec = iraw_v[pl.ds(g * G, G)]
        tot = jnp.zeros((16,), jnp.float32)
        for r in range(G):
            ru = _extract(uvec, r) & 7
            ri = _extract(ivec, r) & 7
            acc = pu_v[slot, r, ru, pl.ds(0, 16)] * qi_v[slot, r, ri, pl.ds(0, 16)]
            for j in range(1, D // 16):
                acc = acc + (pu_v[slot, r, ru, pl.ds(j * 16, 16)] *
                             qi_v[slot, r, ri, pl.ds(j * 16, 16)])
            for idx in rots:
                acc = acc + _permute(acc, idx)
            tot = jnp.where(lanes == r, acc, tot)
        out_v[pl.ds(g * G, G)] = tot

    fire(0, 0)

    def body(h, carry):
        g0 = h * 2
        fire(g0 + 1, 1)
        drain(0)
        compute(g0, 0)

        @pl.when(h < NG // 2 - 1)
        def _():
            fire(g0 + 2, 0)

        drain(1)
        compute(g0 + 1, 1)
        return carry

    lax.fori_loop(0, NG // 2, body, 0)

    pltpu.sync_copy(out_v, out_hbm.at[pl.ds(base, BPW)])


def kernel(u, i, user_emb, item_emb):
    return _mf_sc(u, i, user_emb, item_emb)


# speedup vs baseline: 1.4945x; 1.0018x over previous
import functools

import jax
import jax.numpy as jnp
from jax import lax
from jax.experimental import pallas as pl
from jax.experimental.pallas import tpu as pltpu
from jax.experimental.pallas import tpu_sc as plsc

BATCH = 16384
D = 64
NC = 2
NS = 16
NW = NC * NS
BPW = BATCH // NW
G = 16
NG = BPW // G

_mesh = plsc.VectorSubcoreMesh(core_axis_name="c", subcore_axis_name="s")

_GATHER_DNUMS = lax.GatherDimensionNumbers(
    offset_dims=(), collapsed_slice_dims=(0,), start_index_map=(0,))


def _permute(x, idx):
    return lax.gather(x, idx[:, None], _GATHER_DNUMS, (1,),
                      mode=lax.GatherScatterMode.PROMISE_IN_BOUNDS)


def _extract(v, r):
    return lax.squeeze(lax.slice(v, (r,), (r + 1,)), (0,))


@functools.partial(
    pl.kernel,
    out_type=jax.ShapeDtypeStruct((BATCH,), jnp.float32),
    mesh=_mesh,
    scratch_types=[
        pltpu.VMEM((BPW,), jnp.int32),
        pltpu.VMEM((BPW,), jnp.int32),
        pltpu.VMEM((2, G, 8, D), jnp.float32),
        pltpu.VMEM((2, G, 8, D), jnp.float32),
        pltpu.VMEM((BPW,), jnp.float32),
        pltpu.SemaphoreType.DMA,
        pltpu.SemaphoreType.DMA,
    ],
)
def _mf_sc(u_hbm, i_hbm, ue_hbm, ie_hbm, out_hbm,
           uraw_v, iraw_v, pu_v, qi_v, out_v, sem0, sem1):
    wid = lax.axis_index("s") * NC + lax.axis_index("c")
    base = wid * BPW

    pltpu.sync_copy(u_hbm.at[pl.ds(base, BPW)], uraw_v)
    pltpu.sync_copy(i_hbm.at[pl.ds(base, BPW)], iraw_v)

    sems = (sem0, sem1)
    lanes = lax.iota(jnp.int32, 16)
    rots = [(lanes + s) % 16 for s in (8, 4, 2, 1)]

    def fire(g, slot):
        s = sems[slot]
        ubase = lax.shift_right_logical(uraw_v[pl.ds(g * G, G)], 3) * 8
        ibase = lax.shift_right_logical(iraw_v[pl.ds(g * G, G)], 3) * 8
        for r in range(G):
            ub = pl.multiple_of(_extract(ubase, r), 8)
            ib = pl.multiple_of(_extract(ibase, r), 8)
            pltpu.async_copy(ue_hbm.at[pl.ds(ub, 8), :], pu_v.at[slot, r], s)
            pltpu.async_copy(ie_hbm.at[pl.ds(ib, 8), :], qi_v.at[slot, r], s)

    def drain(slot):
        s = sems[slot]
        for r in range(G):
            pltpu.make_async_copy(ue_hbm.at[pl.ds(0, 8), :], pu_v.at[slot, r], s).wait()
            pltpu.make_async_copy(ie_hbm.at[pl.ds(0, 8), :], qi_v.at[slot, r], s).wait()

    def compute(g, slot):
        usub = uraw_v[pl.ds(g * G, G)] & 7
        isub = iraw_v[pl.ds(g * G, G)] & 7
        tot = jnp.zeros((16,), jnp.float32)
        for r in range(G):
            ru = _extract(usub, r)
            ri = _extract(isub, r)
            acc = pu_v[slot, r, ru, pl.ds(0, 16)] * qi_v[slot, r, ri, pl.ds(0, 16)]
            for j in range(1, D // 16):
                acc = acc + (pu_v[slot, r, ru, pl.ds(j * 16, 16)] *
                             qi_v[slot, r, ri, pl.ds(j * 16, 16)])
            for idx in rots:
                acc = acc + _permute(acc, idx)
            tot = jnp.where(lanes == r, acc, tot)
        out_v[pl.ds(g * G, G)] = tot

    fire(0, 0)

    def body(h, carry):
        g0 = h * 2
        fire(g0 + 1, 1)
        drain(0)
        compute(g0, 0)

        @pl.when(h < NG // 2 - 1)
        def _():
            fire(g0 + 2, 0)

        drain(1)
        compute(g0 + 1, 1)
        return carry

    lax.fori_loop(0, NG // 2, body, 0)

    pltpu.sync_copy(out_v, out_hbm.at[pl.ds(base, BPW)])


def kernel(u, i, user_emb, item_emb):
    return _mf_sc(u, i, user_emb, item_emb)
